# speedup vs baseline: 1.2364x; 1.2364x over previous
import jax
import jax.numpy as jnp
from jax.experimental import pallas as pl
from jax.experimental.pallas import tpu as pltpu

_BM = 200


def _pre_kernel(x_ref, w1_ref, s1_ref):
    s1_ref[...] = jnp.dot(x_ref[...], w1_ref[...],
                          preferred_element_type=jnp.float32)


def _pass1_kernel(adj_ref, s1_ref, w2_ref, h1_ref, hw2_ref, adj_out_ref):
    a = adj_ref[...]
    adj_out_ref[...] = a
    h1 = jnp.maximum(
        jnp.dot(a, s1_ref[...], preferred_element_type=jnp.float32), 0.0)
    h1_ref[...] = h1
    hw2_ref[...] = jnp.dot(h1, w2_ref[...],
                           preferred_element_type=jnp.float32)


def _pass2_kernel(adj_ref, hw2_ref, out_ref):
    x2 = jnp.maximum(
        jnp.dot(adj_ref[...], hw2_ref[...],
                preferred_element_type=jnp.float32), 0.0)
    m = jnp.max(x2, axis=1, keepdims=True)
    e = jnp.exp(x2 - m)
    out_ref[...] = (x2 - m) - jnp.log(jnp.sum(e, axis=1, keepdims=True))


def kernel(x, adj, W1, W2):
    n, nfeat = x.shape
    nhid = W1.shape[1]
    nclass = W2.shape[1]

    s1 = pl.pallas_call(
        _pre_kernel,
        out_shape=jax.ShapeDtypeStruct((n, nhid), jnp.float32),
    )(x, W1)

    grid = (n // _BM,)
    row_blk = lambda i: (i, 0)
    full_blk = lambda i: (0, 0)

    h1, hw2, adj_out = pl.pallas_call(
        _pass1_kernel,
        grid=grid,
        in_specs=[
            pl.BlockSpec((_BM, n), row_blk),
            pl.BlockSpec((n, nhid), full_blk),
            pl.BlockSpec((nhid, nclass), full_blk),
        ],
        out_specs=[
            pl.BlockSpec((_BM, nhid), row_blk),
            pl.BlockSpec((_BM, nclass), row_blk),
            pl.BlockSpec((_BM, n), row_blk),
        ],
        out_shape=[
            jax.ShapeDtypeStruct((n, nhid), jnp.float32),
            jax.ShapeDtypeStruct((n, nclass), jnp.float32),
            jax.ShapeDtypeStruct((n, n), jnp.float32),
        ],
        compiler_params=pltpu.CompilerParams(
            dimension_semantics=("arbitrary",),
        ),
    )(adj, s1, W2)

    logits = pl.pallas_call(
        _pass2_kernel,
        grid=grid,
        in_specs=[
            pl.BlockSpec((_BM, n), row_blk),
            pl.BlockSpec((n, nclass), full_blk),
        ],
        out_specs=pl.BlockSpec((_BM, nclass), row_blk),
        out_shape=jax.ShapeDtypeStruct((n, nclass), jnp.float32),
        compiler_params=pltpu.CompilerParams(
            dimension_semantics=("arbitrary",),
        ),
    )(adj, hw2)

    return (logits, h1, adj_out)
